# Initial kernel scaffold; baseline (speedup 1.0000x reference)
import jax
import jax.numpy as jnp
from jax.experimental import pallas as pl


def kernel(x, edge_index, Wl, bl, Wr, bn_g, bn_b, W1, b1, W2, b2, W3, b3):
    raise NotImplementedError("write your pallas kernel here")


# speedup vs baseline: 6.2685x; 6.2685x over previous
import functools

import jax
import jax.numpy as jnp
from jax import lax
from jax.experimental import pallas as pl
from jax.experimental.pallas import tpu as pltpu
from jax.experimental.pallas import tpu_sc as plsc

N = 10000
E = 320000
D = 128
EPS = 1e-5

NC = 2
NS = 16
NW = NC * NS
CHUNK = 80
EPW = E // NW
NCH = EPW // CHUNK
RPS = N // NS
NPAD = 640 * NS
DW = 16

_mesh = plsc.VectorSubcoreMesh(core_axis_name="c", subcore_axis_name="s")


@functools.partial(
    pl.kernel,
    out_type=jax.ShapeDtypeStruct((NC, NPAD, D), jnp.float32),
    mesh=_mesh,
    scratch_types=[
        pltpu.VMEM((NCH, CHUNK), jnp.int32),
        pltpu.VMEM((NCH, CHUNK), jnp.int32),
        pltpu.VMEM((CHUNK, D), jnp.float32),
        pltpu.VMEM_SHARED((NPAD, D), jnp.float32),
        pltpu.SemaphoreType.DMA,
    ],
)
def _seg_sum(h_hbm, src_hbm, dst_hbm, zeros_hbm, out_hbm,
             src_v, dst_v, rows_v, acc, sem):
    cid = lax.axis_index("c")
    sid = lax.axis_index("s")
    wid = sid * NC + cid

    pltpu.sync_copy(zeros_hbm, acc.at[pl.ds(sid * 640, 640)])
    pltpu.sync_copy(src_hbm.at[wid], src_v)
    pltpu.sync_copy(dst_hbm.at[wid], dst_v)
    plsc.subcore_barrier()

    def body(i, carry):
        pltpu.async_copy(h_hbm.at[src_v.at[i]], rows_v, sem).wait()
        pltpu.sync_copy(rows_v, acc.at[dst_v.at[i]], add=True)
        return carry

    lax.fori_loop(0, NCH, body, 0)
    plsc.subcore_barrier()
    pltpu.sync_copy(acc.at[pl.ds(sid * 640, 640)],
                    out_hbm.at[cid, pl.ds(sid * 640, 640)])


@functools.partial(
    pl.kernel,
    out_type=jax.ShapeDtypeStruct((NC, NPAD, D), jnp.float32),
    mesh=_mesh,
    scratch_types=[
        pltpu.VMEM((NCH, CHUNK), jnp.int32),
        pltpu.VMEM((CHUNK, D), jnp.float32),
        pltpu.VMEM_SHARED((NPAD, D), jnp.float32),
        pltpu.SemaphoreType.DMA,
    ],
)
def _deg_sum(dst_hbm, zeros_hbm, ones_hbm, out_hbm, dst_v, ones_v, acc, sem):
    cid = lax.axis_index("c")
    sid = lax.axis_index("s")
    wid = sid * NC + cid

    pltpu.sync_copy(zeros_hbm, acc.at[pl.ds(sid * 640, 640)])
    pltpu.sync_copy(ones_hbm, ones_v)
    pltpu.sync_copy(dst_hbm.at[wid], dst_v)
    plsc.subcore_barrier()

    def body(i, carry):
        pltpu.sync_copy(ones_v, acc.at[dst_v.at[i]], add=True)
        return carry

    lax.fori_loop(0, NCH, body, 0)
    plsc.subcore_barrier()
    pltpu.sync_copy(acc.at[pl.ds(sid * 640, 640)],
                    out_hbm.at[cid, pl.ds(sid * 640, 640)])


def _inv_deg_body(degp_ref, out_ref):
    deg = degp_ref[0, 0:N] + degp_ref[1, 0:N]
    out_ref[...] = 1.0 / jnp.maximum(deg, 1.0)


def _layer_body(p_ref, invd_ref, h_ref, wl_ref, bl_ref, wr_ref, g_ref, b_ref,
                out_ref):
    agg = (p_ref[0, 0:N] + p_ref[1, 0:N]) * invd_ref[...]
    dn = (((1,), (1,)), ((), ()))
    h = (lax.dot_general(agg, wl_ref[...], dn, preferred_element_type=jnp.float32)
         + bl_ref[...]
         + lax.dot_general(h_ref[...], wr_ref[...], dn,
                           preferred_element_type=jnp.float32))
    mean = jnp.mean(h, axis=0, keepdims=True)
    var = jnp.mean((h - mean) ** 2, axis=0, keepdims=True)
    h = (h - mean) / jnp.sqrt(var + EPS) * g_ref[...] + b_ref[...]
    out_ref[...] = jnp.maximum(h, 0.0)


def _readout_body(x_ref, h1_ref, h2_ref, h3_ref, w1_ref, b1_ref, w2_ref,
                  b2_ref, w3_ref, b3_ref, out_ref):
    node = x_ref[...] + h1_ref[...] + h2_ref[...] + h3_ref[...]
    hg = jnp.sum(node, axis=0, keepdims=True)
    dn = (((1,), (1,)), ((), ()))
    t = lax.dot_general(hg, w1_ref[...], dn, preferred_element_type=jnp.float32)
    t = jnp.maximum(t + b1_ref[...], 0.0)
    t = lax.dot_general(t, w2_ref[...], dn, preferred_element_type=jnp.float32)
    t = jnp.maximum(t + b2_ref[...], 0.0)
    s = jnp.sum(t * w3_ref[...], axis=1, keepdims=True)
    out_ref[...] = s + b3_ref[...]


def _inv_deg_call(degp):
    return pl.pallas_call(
        _inv_deg_body,
        out_shape=jax.ShapeDtypeStruct((N, D), jnp.float32),
    )(degp)


def _layer_call(p, invd, h, wl, blv, wr, g, b):
    return pl.pallas_call(
        _layer_body,
        out_shape=jax.ShapeDtypeStruct((N, D), jnp.float32),
    )(p, invd, h, wl, blv, wr, g, b)


def _readout_call(x, h1, h2, h3, w1, b1v, w2, b2v, w3, b3v):
    return pl.pallas_call(
        _readout_body,
        out_shape=jax.ShapeDtypeStruct((1, 1), jnp.float32),
    )(x, h1, h2, h3, w1, b1v, w2, b2v, w3, b3v)


def kernel(x, edge_index, Wl, bl, Wr, bn_g, bn_b, W1, b1, W2, b2, W3, b3):
    src = edge_index[0].reshape(NW, NCH, CHUNK)
    dst = edge_index[1].reshape(NW, NCH, CHUNK)
    zeros_d = jnp.zeros((640, D), jnp.float32)
    ones_w = jnp.ones((CHUNK, D), jnp.float32)

    degp = _deg_sum(dst, zeros_d, ones_w)
    invd = _inv_deg_call(degp)

    h = x
    hs = [x]
    for l in range(3):
        p = _seg_sum(h, src, dst, zeros_d)
        h = _layer_call(p, invd, h, Wl[l], bl[l][None], Wr[l],
                        bn_g[l][None], bn_b[l][None])
        hs.append(h)

    return _readout_call(x, hs[1], hs[2], hs[3], W1, b1[None], W2,
                         b2[None], W3, b3[None])
